# baseline (speedup 1.0000x reference)
import numpy as np
import jax
import jax.numpy as jnp
from jax.experimental import pallas as pl
from jax.experimental.pallas import tpu as pltpu

_Hl, _Wl = 14, 14
_SCALE = 16
_R_MAX = 3
_ALPHA_DYN = 2.0
_Hh, _Wh = _Hl * _SCALE, _Wl * _SCALE
_NPIX = _Hh * _Wh
_EXT = _Wl + 2 * _R_MAX
_NR = 7 * _EXT
_NRP = 144
_P = _SCALE * _Wh
_NCELL = _Hl * _NRP
_NMAP = 8
_NFA = 104
_BPS = 7
_LOG2E = float(np.log2(np.e))


def _resize_mat(dst, src):
    m = np.zeros((dst, src), np.float32)
    for y in range(dst):
        u = (y + 0.5) * src / dst - 0.5
        i0 = int(np.floor(u))
        f = u - i0
        m[y, min(max(i0, 0), src - 1)] += 1.0 - f
        m[y, min(max(i0 + 1, 0), src - 1)] += f
    return m


def _build_static():
    dys = np.arange(-_R_MAX, _R_MAX + 1)
    ext_j = np.arange(-_R_MAX, _Wl + _R_MAX)
    ts = np.arange(_Hl)
    iu = np.broadcast_to(ts[:, None, None] + dys[None, :, None],
                         (_Hl, 7, _EXT)).reshape(_Hl, _NR)
    ju = np.broadcast_to(ext_j[None, None, :],
                         (_Hl, 7, _EXT)).reshape(_Hl, _NR)
    npad = _NRP - _NR
    padi = np.full((_Hl, npad), 10 ** 4, np.int64)
    iu = np.concatenate([iu, padi], 1)
    ju = np.concatenate([ju, padi], 1)
    icl = np.clip(iu, 0, _Hl - 1)
    jcl = np.clip(ju, 0, _Wl - 1)
    live = np.zeros((_Hl, _NRP), bool)
    live[:, :_NR] = True
    fl_iu = iu.reshape(-1)
    fl_ju = ju.reshape(-1)
    fl_ic = icl.reshape(-1)
    fl_jc = jcl.reshape(-1)
    fl_live = live.reshape(-1)

    ohit = np.zeros((_Hl, _NCELL), np.float32)
    ohjt = np.zeros((_Wl, _NCELL), np.float32)
    r = np.arange(_NCELL)[fl_live]
    ohit[fl_ic[fl_live], r] = 1.0
    ohjt[fl_jc[fl_live], r] = 1.0

    band = np.repeat(ts, _NRP)
    cxv = (fl_jc + 0.5) * _SCALE - 0.5 - 112.0
    cyl = (fl_ic + 0.5) * _SCALE - 0.5 - _SCALE * band
    sqc = (fl_iu - band).astype(np.float64) ** 2 + fl_ju.astype(
        np.float64) ** 2
    sqc = np.where(fl_live, sqc, 1e8)
    jm2 = np.where(fl_live, -2.0 * fl_ju, 0.0)

    wm = np.zeros((_NMAP, 16, _NCELL), np.float64)
    wm[0, 0] = -cxv * cxv
    wm[0, 1] = 2.0 * cxv
    wm[0, 4] = -1.0
    wm[1, 0] = -cxv * cyl
    wm[1, 1] = cyl
    wm[1, 2] = cxv
    wm[1, 5] = -1.0
    wm[2, 0] = -cyl * cyl
    wm[2, 2] = 2.0 * cyl
    wm[2, 6] = -1.0
    wm[3, 10] = -1.0
    wm[4, 0] = -1.0
    wm[5, 7] = 1.0
    wm[6, 8] = 1.0
    wm[7, 9] = 1.0
    wmts = wm.reshape(_NMAP * 16, _NCELL).astype(np.float32)

    dy2q = np.where(fl_live, (fl_iu - band).astype(np.float64) ** 2, 0.0)
    ju2q = np.where(fl_live, fl_ju.astype(np.float64) ** 2, 0.0)
    rst = np.zeros((16, _NCELL), np.float64)
    rst[0] = np.where(fl_live, dy2q / 4.0, float(2 ** 20))
    rst[3] = jm2 / 4.0
    rst[11] = 0.25
    rst[12] = -0.25
    rst[13] = ju2q / 4.0
    rst = rst.astype(np.float32).reshape(16, _Hl, _NRP).transpose(1, 0, 2)

    flat = fl_ic * _Wl + fl_jc
    ohf = np.zeros((_Hl, _Hl * _Wl, _NRP), np.float32)
    cc = np.tile(np.arange(_NRP), _Hl)
    ohf[band[fl_live], flat[fl_live], cc[fl_live]] = 1.0

    p = np.arange(_P)
    xg = (p % _Wh).astype(np.float64)
    ylv = (p // _Wh).astype(np.float64)
    xv = xg - 112.0
    vcf = np.floor(xg / _SCALE)
    cfk = np.stack([np.ones(_P), xv, ylv, vcf, xv * xv, xv * ylv, ylv * ylv,
                    vcf * vcf, np.ones(_P), np.zeros(_P),
                    np.zeros(_P)]).astype(np.float32)

    bh14 = _resize_mat(_Hl, _Hh)
    bwt = _resize_mat(_Wl, _Wh).T
    ah = _resize_mat(_Hh, _Hl)
    aht = ah.T.copy()
    ahb = np.zeros((_Hl, _Hl, _P), np.float32)
    for g in range(_Hl):
        ahb[g] = ah[16 * g + p // _Wh, :].T
    return ohit, ohjt, wmts, rst, ohf, cfk, bh14, bwt, aht, ahb


(_OHIT, _OHJT, _WMTS, _RST, _OHF, _CFK, _BH14, _BWT, _AHT,
 _AHB) = _build_static()


def _jbu_tile(guide3_ref, g672_ref, sxr_ref, syr_ref, thr_ref, srr_ref,
              feat_ref, ohf_ref, ohit_ref, ohjt_ref, wmts_ref, rst_ref,
              cfk_ref, bh_ref, bwt_ref, aht_ref, ahb_ref, out_ref,
              rl_scr, fb_scr, tt_scr):
    g = pl.program_id(0)
    f32 = jnp.float32
    hi = jax.lax.Precision.HIGHEST

    @pl.when(g == 0)
    def _build_tables():
        sxt = sxr_ref[...].T
        syt = syr_ref[...].T
        tht = thr_ref[...].T
        srt = srr_ref[...].T
        sxm = jnp.maximum(jnp.exp(sxt), 1e-6)
        sym = jnp.maximum(jnp.exp(syt), 1e-6)
        srm = jnp.maximum(jnp.exp(srt), 1e-6)
        isx = _LOG2E / (2.0 * sxm * sxm + 1e-8)
        isy = _LOG2E / (2.0 * sym * sym + 1e-8)
        isr = _LOG2E / (2.0 * srm * srm + 1e-8)
        th = jnp.pi * jnp.tanh(tht)
        ct = jnp.cos(th)
        st = jnp.sin(th)
        qa = ct * ct * isx + st * st * isy
        qb = 2.0 * ct * st * (isx - isy)
        qc = st * st * isx + ct * ct * isy

        bf = jnp.bfloat16

        def dot2(a, b):
            ah_ = a.astype(bf)
            al_ = (a - ah_.astype(f32)).astype(bf)
            return (jnp.dot(ah_, b, preferred_element_type=f32)
                    + jnp.dot(al_, b, preferred_element_type=f32))

        bh = bh_ref[...].astype(bf)
        bwt = bwt_ref[...].astype(bf)
        glt = []
        for ch in range(3):
            gc = g672_ref[ch * _Hh:(ch + 1) * _Hh, :]
            glr = dot2(dot2(gc, bwt).T, bh.T).T
            glt.append(glr.T)
        glsq = glt[0] * glt[0] + glt[1] * glt[1] + glt[2] * glt[2]
        maps = [qa, qb, qc, isr, isr * glsq,
                2.0 * isr * glt[0], 2.0 * isr * glt[1], 2.0 * isr * glt[2]]

        ohit = ohit_ref[...].astype(bf)
        ohjt = ohjt_ref[...]
        rlt = jnp.zeros((16, _NCELL), f32)
        for k in range(_NMAP):
            t = dot2(maps[k], ohit)
            gk = jnp.sum(t * ohjt, axis=0, keepdims=True)
            rlt = rlt + wmts_ref[16 * k:16 * (k + 1), :] * gk
        for t in range(_Hl):
            rl_scr[t, :, :] = rlt[:, _NRP * t:_NRP * (t + 1)]

        fa = jnp.concatenate(
            [feat_ref[...], jnp.zeros((1, _Hl * _Wl), f32) + 1.0,
             jnp.zeros((_NFA - 97, _Hl * _Wl), f32)],
            axis=0).astype(jnp.bfloat16)
        ohfb = ohf_ref[...].astype(jnp.bfloat16)
        for t in range(_Hl):
            fb_scr[t, :, :] = jnp.dot(fa, ohfb[t],
                                      preferred_element_type=f32
                                      ).astype(jnp.bfloat16)

        smax = jnp.exp(jnp.maximum(sxr_ref[...], syr_ref[...]))
        tsig = jnp.dot(smax, aht_ref[...], preferred_element_type=f32,
                       precision=hi)
        tt_scr[...] = jnp.concatenate([tsig] * _SCALE, axis=1)

    f32 = jnp.float32
    bf16 = jnp.bfloat16
    dn = (((0,), (0,)), ((), ()))
    ones14 = jnp.zeros((1, _Hl), f32) + 1.0
    cfk = cfk_ref[...]
    for h in range(_BPS):
        band = _BPS * g + h
        sl_ = slice(h * _P, (h + 1) * _P)
        sig = jnp.dot(ones14, ahb_ref[h] * tt_scr[...],
                      preferred_element_type=f32,
                      precision=jax.lax.Precision.HIGHEST)
        rm = jnp.clip(jnp.ceil(_ALPHA_DYN * sig), 1, _R_MAX)
        r2 = rm * rm

        gh = guide3_ref[:, sl_]
        gh0 = gh[0:1, :]
        gh1 = gh[1:2, :]
        gh2 = gh[2:3, :]
        ghsq = gh0 * gh0 + gh1 * gh1 + gh2 * gh2
        cf = jnp.concatenate([
            cfk[0:7, :], gh, ghsq, cfk[7:8, :], r2, cfk[8:11, :]], axis=0)
        cfh = cf.astype(bf16)
        cfl = (cf - cfh.astype(f32)).astype(bf16)
        rl = rl_scr[band]
        rlh = rl.astype(bf16)
        rll = (rl - rlh.astype(f32)).astype(bf16)
        zb = jnp.zeros((16, _NRP), bf16)
        rs = rst_ref[h].reshape(16, _NRP).astype(bf16)
        lhs = jnp.concatenate(
            [jnp.concatenate([rlh, rlh, rll], axis=0),
             jnp.concatenate([rs, zb, zb], axis=0)], axis=1)
        rhs = jnp.concatenate([cfh, cfl, cfh], axis=0)
        both = jax.lax.dot_general(lhs, rhs, dn, preferred_element_type=f32)
        lw2 = both[0:_NRP, :]
        sqmr = both[_NRP:2 * _NRP, :]
        lwm = lw2 - jnp.maximum(sqmr - 0.125, 0.0) * 4e30
        m = jnp.max(lwm, axis=0, keepdims=True)
        s = jnp.exp2(lwm - m)
        sh = s.astype(bf16)
        numa = jax.lax.dot_general(fb_scr[band], sh,
                                   (((1,), (0,)), ((), ())),
                                   preferred_element_type=f32)
        out_ref[:, sl_] = numa[0:96, :] * (1.0 / numa[96:97, :])


def kernel(feat_lr, guide_hr, sx_raw, sy_raw, th_raw, sr_raw):
    f32 = jnp.float32
    nc = feat_lr.shape[1]

    guide3 = guide_hr[0].astype(f32).reshape(3, _NPIX)
    g672 = guide_hr[0].astype(f32).reshape(3 * _Hh, _Wh)
    feat196 = feat_lr[0].astype(f32).reshape(nc, _Hl * _Wl)

    full = lambda g: (0, 0)
    out = pl.pallas_call(
        _jbu_tile,
        grid=(_Hl // _BPS,),
        in_specs=[
            pl.BlockSpec((3, _BPS * _P), lambda g: (0, g)),
            pl.BlockSpec((3 * _Hh, _Wh), full),
            pl.BlockSpec((_Hl, _Wl), full),
            pl.BlockSpec((_Hl, _Wl), full),
            pl.BlockSpec((_Hl, _Wl), full),
            pl.BlockSpec((_Hl, _Wl), full),
            pl.BlockSpec((nc, _Hl * _Wl), full),
            pl.BlockSpec((_Hl, _Hl * _Wl, _NRP), lambda g: (0, 0, 0)),
            pl.BlockSpec((_Hl, _NCELL), full),
            pl.BlockSpec((_Wl, _NCELL), full),
            pl.BlockSpec((_NMAP * 16, _NCELL), full),
            pl.BlockSpec((_BPS, 16, _NRP), lambda g: (g, 0, 0)),
            pl.BlockSpec((11, _P), full),
            pl.BlockSpec((_Hl, _Hh), full),
            pl.BlockSpec((_Hh, _Wl), full),
            pl.BlockSpec((_Hl, _Hh), full),
            pl.BlockSpec((_BPS, _Hl, _P), lambda g: (g, 0, 0)),
        ],
        out_specs=pl.BlockSpec((nc, _BPS * _P), lambda g: (0, g)),
        out_shape=jax.ShapeDtypeStruct((nc, _NPIX), f32),
        scratch_shapes=[
            pltpu.VMEM((_Hl, 16, _NRP), f32),
            pltpu.VMEM((_Hl, _NFA, _NRP), jnp.bfloat16),
            pltpu.VMEM((_Hl, _P), f32),
        ],
    )(guide3, g672, sx_raw[0, 0], sy_raw[0, 0], th_raw[0, 0], sr_raw[0, 0],
      feat196, jnp.asarray(_OHF), jnp.asarray(_OHIT), jnp.asarray(_OHJT),
      jnp.asarray(_WMTS), jnp.asarray(_RST), jnp.asarray(_CFK),
      jnp.asarray(_BH14), jnp.asarray(_BWT), jnp.asarray(_AHT),
      jnp.asarray(_AHB))

    return out.reshape(1, nc, _Hh, _Wh).astype(feat_lr.dtype)


# speedup vs baseline: 1.0337x; 1.0337x over previous
import numpy as np
import jax
import jax.numpy as jnp
from jax.experimental import pallas as pl
from jax.experimental.pallas import tpu as pltpu

_Hl, _Wl = 14, 14
_SCALE = 16
_R_MAX = 3
_ALPHA_DYN = 2.0
_Hh, _Wh = _Hl * _SCALE, _Wl * _SCALE
_NPIX = _Hh * _Wh
_EXT = _Wl + 2 * _R_MAX
_NR = 7 * _EXT
_NRP = 144
_P = _SCALE * _Wh
_NCELL = _Hl * _NRP
_NMAP = 8
_NFA = 104
_LOG2E = float(np.log2(np.e))


def _resize_mat(dst, src):
    m = np.zeros((dst, src), np.float32)
    for y in range(dst):
        u = (y + 0.5) * src / dst - 0.5
        i0 = int(np.floor(u))
        f = u - i0
        m[y, min(max(i0, 0), src - 1)] += 1.0 - f
        m[y, min(max(i0 + 1, 0), src - 1)] += f
    return m


def _build_static():
    dys = np.arange(-_R_MAX, _R_MAX + 1)
    ext_j = np.arange(-_R_MAX, _Wl + _R_MAX)
    ts = np.arange(_Hl)
    iu = np.broadcast_to(ts[:, None, None] + dys[None, :, None],
                         (_Hl, 7, _EXT)).reshape(_Hl, _NR)
    ju = np.broadcast_to(ext_j[None, None, :],
                         (_Hl, 7, _EXT)).reshape(_Hl, _NR)
    npad = _NRP - _NR
    padi = np.full((_Hl, npad), 10 ** 4, np.int64)
    iu = np.concatenate([iu, padi], 1)
    ju = np.concatenate([ju, padi], 1)
    icl = np.clip(iu, 0, _Hl - 1)
    jcl = np.clip(ju, 0, _Wl - 1)
    live = np.zeros((_Hl, _NRP), bool)
    live[:, :_NR] = True
    fl_iu = iu.reshape(-1)
    fl_ju = ju.reshape(-1)
    fl_ic = icl.reshape(-1)
    fl_jc = jcl.reshape(-1)
    fl_live = live.reshape(-1)

    ohit = np.zeros((_Hl, _NCELL), np.float32)
    ohjt = np.zeros((_Wl, _NCELL), np.float32)
    r = np.arange(_NCELL)[fl_live]
    ohit[fl_ic[fl_live], r] = 1.0
    ohjt[fl_jc[fl_live], r] = 1.0

    band = np.repeat(ts, _NRP)
    cxv = (fl_jc + 0.5) * _SCALE - 0.5 - 112.0
    cyl = (fl_ic + 0.5) * _SCALE - 0.5 - _SCALE * band
    sqc = (fl_iu - band).astype(np.float64) ** 2 + fl_ju.astype(
        np.float64) ** 2
    sqc = np.where(fl_live, sqc, 1e8)
    jm2 = np.where(fl_live, -2.0 * fl_ju, 0.0)

    wm = np.zeros((_NMAP, 16, _NCELL), np.float64)
    wm[0, 0] = -cxv * cxv
    wm[0, 1] = 2.0 * cxv
    wm[0, 4] = -1.0
    wm[1, 0] = -cxv * cyl
    wm[1, 1] = cyl
    wm[1, 2] = cxv
    wm[1, 5] = -1.0
    wm[2, 0] = -cyl * cyl
    wm[2, 2] = 2.0 * cyl
    wm[2, 6] = -1.0
    wm[3, 10] = -1.0
    wm[4, 0] = -1.0
    wm[5, 7] = 1.0
    wm[6, 8] = 1.0
    wm[7, 9] = 1.0
    wmts = wm.reshape(_NMAP * 16, _NCELL).astype(np.float32)

    dy2q = np.where(fl_live, (fl_iu - band).astype(np.float64) ** 2, 0.0)
    ju2q = np.where(fl_live, fl_ju.astype(np.float64) ** 2, 0.0)
    rst = np.zeros((16, _NCELL), np.float64)
    rst[0] = np.where(fl_live, dy2q / 4.0, float(2 ** 20))
    rst[3] = jm2 / 4.0
    rst[11] = 0.25
    rst[12] = -0.25
    rst[13] = ju2q / 4.0
    rst = rst.astype(np.float32).reshape(16, _Hl, _NRP).transpose(1, 0, 2)

    flat = fl_ic * _Wl + fl_jc
    ohf = np.zeros((_Hl, _Hl * _Wl, _NRP), np.float32)
    cc = np.tile(np.arange(_NRP), _Hl)
    ohf[band[fl_live], flat[fl_live], cc[fl_live]] = 1.0

    p = np.arange(_P)
    xg = (p % _Wh).astype(np.float64)
    ylv = (p // _Wh).astype(np.float64)
    xv = xg - 112.0
    vcf = np.floor(xg / _SCALE)
    cfk = np.stack([np.ones(_P), xv, ylv, vcf, xv * xv, xv * ylv, ylv * ylv,
                    vcf * vcf, np.ones(_P), np.zeros(_P),
                    np.zeros(_P)]).astype(np.float32)

    bh14 = _resize_mat(_Hl, _Hh)
    bwt = _resize_mat(_Wl, _Wh).T
    ah = _resize_mat(_Hh, _Hl)
    aht = ah.T.copy()
    ahb = np.zeros((_Hl, _Hl, _P), np.float32)
    for g in range(_Hl):
        ahb[g] = ah[16 * g + p // _Wh, :].T
    return ohit, ohjt, wmts, rst, ohf, cfk, bh14, bwt, aht, ahb


(_OHIT, _OHJT, _WMTS, _RST, _OHF, _CFK, _BH14, _BWT, _AHT,
 _AHB) = _build_static()


def _jbu_tile(guide3_ref, g672_ref, sxr_ref, syr_ref, thr_ref, srr_ref,
              feat_ref, ohf_ref, ohit_ref, ohjt_ref, wmts_ref, rst_ref,
              cfk_ref, bh_ref, bwt_ref, aht_ref, ahb_ref, out_ref,
              rl_scr, fb_scr, tt_scr):
    g = pl.program_id(0)
    f32 = jnp.float32
    hi = jax.lax.Precision.HIGHEST

    @pl.when(g == 0)
    def _build_tables():
        sxt = sxr_ref[...].T
        syt = syr_ref[...].T
        tht = thr_ref[...].T
        srt = srr_ref[...].T
        sxm = jnp.maximum(jnp.exp(sxt), 1e-6)
        sym = jnp.maximum(jnp.exp(syt), 1e-6)
        srm = jnp.maximum(jnp.exp(srt), 1e-6)
        isx = _LOG2E / (2.0 * sxm * sxm + 1e-8)
        isy = _LOG2E / (2.0 * sym * sym + 1e-8)
        isr = _LOG2E / (2.0 * srm * srm + 1e-8)
        th = jnp.pi * jnp.tanh(tht)
        ct = jnp.cos(th)
        st = jnp.sin(th)
        qa = ct * ct * isx + st * st * isy
        qb = 2.0 * ct * st * (isx - isy)
        qc = st * st * isx + ct * ct * isy

        bf = jnp.bfloat16

        def dot2(a, b):
            ah_ = a.astype(bf)
            al_ = (a - ah_.astype(f32)).astype(bf)
            return (jnp.dot(ah_, b, preferred_element_type=f32)
                    + jnp.dot(al_, b, preferred_element_type=f32))

        bh = bh_ref[...].astype(bf)
        bwt = bwt_ref[...].astype(bf)
        glt = []
        for ch in range(3):
            gc = g672_ref[ch * _Hh:(ch + 1) * _Hh, :]
            glr = dot2(dot2(gc, bwt).T, bh.T).T
            glt.append(glr.T)
        glsq = glt[0] * glt[0] + glt[1] * glt[1] + glt[2] * glt[2]
        maps = [qa, qb, qc, isr, isr * glsq,
                2.0 * isr * glt[0], 2.0 * isr * glt[1], 2.0 * isr * glt[2]]

        ohit = ohit_ref[...].astype(bf)
        ohjt = ohjt_ref[...]
        rlt = jnp.zeros((16, _NCELL), f32)
        for k in range(_NMAP):
            t = dot2(maps[k], ohit)
            gk = jnp.sum(t * ohjt, axis=0, keepdims=True)
            rlt = rlt + wmts_ref[16 * k:16 * (k + 1), :] * gk
        for t in range(_Hl):
            rl_scr[t, :, :] = rlt[:, _NRP * t:_NRP * (t + 1)]

        fa = jnp.concatenate(
            [feat_ref[...], jnp.zeros((1, _Hl * _Wl), f32) + 1.0,
             jnp.zeros((_NFA - 97, _Hl * _Wl), f32)],
            axis=0).astype(jnp.bfloat16)
        ohfb = ohf_ref[...].astype(jnp.bfloat16)
        for t in range(_Hl):
            fb_scr[t, :, :] = jnp.dot(fa, ohfb[t],
                                      preferred_element_type=f32
                                      ).astype(jnp.bfloat16)

        smax = jnp.exp(jnp.maximum(sxr_ref[...], syr_ref[...]))
        tsig = jnp.dot(smax, aht_ref[...], preferred_element_type=f32,
                       precision=hi)
        tt_scr[...] = jnp.concatenate([tsig] * _SCALE, axis=1)

    f32 = jnp.float32
    bf16 = jnp.bfloat16
    dn = (((0,), (0,)), ((), ()))
    ones14 = jnp.zeros((1, _Hl), f32) + 1.0
    cfk = cfk_ref[...]
    for h in range(2):
        band = 2 * g + h
        sl_ = slice(h * _P, (h + 1) * _P)
        sig = jnp.dot(ones14, ahb_ref[h] * tt_scr[...],
                      preferred_element_type=f32,
                      precision=jax.lax.Precision.HIGHEST)
        rm = jnp.clip(jnp.ceil(_ALPHA_DYN * sig), 1, _R_MAX)
        r2 = rm * rm

        gh = guide3_ref[:, sl_]
        gh0 = gh[0:1, :]
        gh1 = gh[1:2, :]
        gh2 = gh[2:3, :]
        ghsq = gh0 * gh0 + gh1 * gh1 + gh2 * gh2
        cf = jnp.concatenate([
            cfk[0:7, :], gh, ghsq, cfk[7:8, :], r2, cfk[8:11, :]], axis=0)
        cfh = cf.astype(bf16)
        cfl = (cf - cfh.astype(f32)).astype(bf16)
        rl = rl_scr[band]
        rlh = rl.astype(bf16)
        rll = (rl - rlh.astype(f32)).astype(bf16)
        zb = jnp.zeros((16, _NRP), bf16)
        rs = rst_ref[h].reshape(16, _NRP).astype(bf16)
        lhs = jnp.concatenate(
            [jnp.concatenate([rlh, rlh, rll], axis=0),
             jnp.concatenate([rs, zb, zb], axis=0)], axis=1)
        rhs = jnp.concatenate([cfh, cfl, cfh], axis=0)
        both = jax.lax.dot_general(lhs, rhs, dn, preferred_element_type=f32)
        lw2 = both[0:_NRP, :]
        sqmr = both[_NRP:2 * _NRP, :]
        lwm = lw2 - jnp.maximum(sqmr - 0.125, 0.0) * 4e30
        m = jnp.max(lwm, axis=0, keepdims=True)
        s = jnp.exp2(lwm - m)
        sh = s.astype(bf16)
        numa = jax.lax.dot_general(fb_scr[band], sh,
                                   (((1,), (0,)), ((), ())),
                                   preferred_element_type=f32)
        out_ref[:, sl_] = numa[0:96, :] * (1.0 / numa[96:97, :])


def kernel(feat_lr, guide_hr, sx_raw, sy_raw, th_raw, sr_raw):
    f32 = jnp.float32
    nc = feat_lr.shape[1]

    guide3 = guide_hr[0].astype(f32).reshape(3, _NPIX)
    g672 = guide_hr[0].astype(f32).reshape(3 * _Hh, _Wh)
    feat196 = feat_lr[0].astype(f32).reshape(nc, _Hl * _Wl)

    full = lambda g: (0, 0)
    out = pl.pallas_call(
        _jbu_tile,
        grid=(_Hl // 2,),
        in_specs=[
            pl.BlockSpec((3, 2 * _P), lambda g: (0, g)),
            pl.BlockSpec((3 * _Hh, _Wh), full),
            pl.BlockSpec((_Hl, _Wl), full),
            pl.BlockSpec((_Hl, _Wl), full),
            pl.BlockSpec((_Hl, _Wl), full),
            pl.BlockSpec((_Hl, _Wl), full),
            pl.BlockSpec((nc, _Hl * _Wl), full),
            pl.BlockSpec((_Hl, _Hl * _Wl, _NRP), lambda g: (0, 0, 0)),
            pl.BlockSpec((_Hl, _NCELL), full),
            pl.BlockSpec((_Wl, _NCELL), full),
            pl.BlockSpec((_NMAP * 16, _NCELL), full),
            pl.BlockSpec((2, 16, _NRP), lambda g: (g, 0, 0)),
            pl.BlockSpec((11, _P), full),
            pl.BlockSpec((_Hl, _Hh), full),
            pl.BlockSpec((_Hh, _Wl), full),
            pl.BlockSpec((_Hl, _Hh), full),
            pl.BlockSpec((2, _Hl, _P), lambda g: (g, 0, 0)),
        ],
        out_specs=pl.BlockSpec((nc, 2 * _P), lambda g: (0, g)),
        out_shape=jax.ShapeDtypeStruct((nc, _NPIX), f32),
        scratch_shapes=[
            pltpu.VMEM((_Hl, 16, _NRP), f32),
            pltpu.VMEM((_Hl, _NFA, _NRP), jnp.bfloat16),
            pltpu.VMEM((_Hl, _P), f32),
        ],
    )(guide3, g672, sx_raw[0, 0], sy_raw[0, 0], th_raw[0, 0], sr_raw[0, 0],
      feat196, jnp.asarray(_OHF), jnp.asarray(_OHIT), jnp.asarray(_OHJT),
      jnp.asarray(_WMTS), jnp.asarray(_RST), jnp.asarray(_CFK),
      jnp.asarray(_BH14), jnp.asarray(_BWT), jnp.asarray(_AHT),
      jnp.asarray(_AHB))

    return out.reshape(1, nc, _Hh, _Wh).astype(feat_lr.dtype)
